# baseline (speedup 1.0000x reference)
import functools

import jax
import jax.numpy as jnp
from jax import lax
from jax.experimental import pallas as pl
from jax.experimental.pallas import tpu as pltpu
from jax.experimental.pallas import tpu_sc as plsc

N = 10000
E = 320000
D = 128
NTILES = 16
EPT = E // NTILES
K = 80
NCH = EPT // K
ACCR = 10240
ZPT = ACCR // NTILES
CPT = N // NTILES
_LN2 = 0.6931471805599453


def _ssp(t):
    return jnp.maximum(t, 0.0) + jnp.log1p(jnp.exp(-jnp.abs(t))) - _LN2


def _proj_body(a_ref, w_ref, o_ref):
    o_ref[...] = jnp.dot(a_ref[0], w_ref[0],
                         preferred_element_type=jnp.float32)[None]


def _filter_body(f_ref, rc_ref, w1_ref, b1_ref, w2_ref, b2_ref, o_ref):
    h = _ssp(jnp.dot(f_ref[...], w1_ref[...],
                     preferred_element_type=jnp.float32) + b1_ref[...])
    w = jnp.dot(h, w2_ref[...], preferred_element_type=jnp.float32) + b2_ref[...]
    o_ref[...] = w * rc_ref[...]


def _out_body(c_ref, w1_ref, b1_ref, w2_ref, b2_ref, o_ref):
    h = _ssp(jnp.dot(c_ref[0], w1_ref[0],
                     preferred_element_type=jnp.float32) + b1_ref[0])
    o_ref[...] = (jnp.dot(h, w2_ref[0],
                          preferred_element_type=jnp.float32) + b2_ref[0])[None]


_sc_mesh = plsc.VectorSubcoreMesh(core_axis_name="c", subcore_axis_name="s")


@functools.partial(
    pl.kernel,
    mesh=_sc_mesh,
    out_type=jax.ShapeDtypeStruct((2, ACCR, D), jnp.float32),
    scratch_types=[
        pltpu.VMEM((2, 2, K), jnp.int32),
        pltpu.VMEM((K, D), jnp.float32),
        pltpu.VMEM((K, D), jnp.float32),
        pltpu.VMEM_SHARED((ACCR, D), jnp.float32),
        pltpu.SemaphoreType.DMA,
        pltpu.SemaphoreType.DMA,
        pltpu.SemaphoreType.DMA,
    ],
)
def _sc_conv(table, cidx, wij, zeros, out,
             cidx_v, wij_v, rows_v, acc,
             sem0, sem1, sem2):
    c = lax.axis_index("c")
    s = lax.axis_index("s")
    pltpu.sync_copy(zeros, acc.at[pl.ds(s * ZPT, ZPT)])
    pltpu.sync_copy(cidx.at[c, s, 0], cidx_v.at[0])
    plsc.subcore_barrier()

    ebase = s * EPT

    def chunk(k, carry):
        buf = lax.rem(k, 2)
        nbuf = 1 - buf
        kn = jnp.minimum(k + 1, NCH - 1)
        hpre = pltpu.async_copy(cidx.at[c, s, kn], cidx_v.at[nbuf], sem2)
        row0 = ebase + k * K
        hw = pltpu.async_copy(wij.at[pl.ds(row0, K)], wij_v, sem1)
        hg = pltpu.async_copy(table.at[cidx_v.at[buf, 0]], rows_v, sem0)
        hg.wait()
        hw.wait()

        for e in range(K):
            for dd in range(D // 16):
                sl = pl.ds(dd * 16, 16)
                rows_v[e, sl] = rows_v[e, sl] * wij_v[e, sl]
        pltpu.sync_copy(rows_v, acc.at[cidx_v.at[buf, 1]], add=True)
        hpre.wait()
        return carry

    lax.fori_loop(0, NCH, chunk, 0)
    plsc.subcore_barrier()
    pltpu.sync_copy(acc.at[pl.ds(s * ZPT, ZPT)],
                    out.at[c, pl.ds(s * ZPT, ZPT)])


def kernel(x, y, f_ij, idx_i, idx_j, rcut_ij,
           W_in2f, W_in2f_y, fw1, fb1, fw2, fb2,
           ow1, ob1, ow2, ob2, oyw1, oyb1, oyw2, oyb2):
    perm = (jnp.arange(NTILES)[:, None, None] * EPT
            + jnp.arange(NCH)[None, :, None]
            + jnp.arange(K)[None, None, :] * NCH).reshape(E)
    f_ij = f_ij[perm]
    rcut_ij = rcut_ij[perm]
    ii = idx_i[perm].astype(jnp.int32)
    jj = idx_j[perm].astype(jnp.int32)

    BN = 2000
    ab = jnp.stack([y, x])
    Wb = jnp.stack([W_in2f_y, W_in2f])
    table = pl.pallas_call(
        _proj_body,
        grid=(2, N // BN),
        in_specs=[pl.BlockSpec((1, BN, D), lambda g, i: (g, i, 0)),
                  pl.BlockSpec((1, D, D), lambda g, i: (g, 0, 0))],
        out_specs=pl.BlockSpec((1, BN, D), lambda g, i: (g, i, 0)),
        out_shape=jax.ShapeDtypeStruct((2, N, D), jnp.float32),
    )(ab, Wb).reshape(2 * N, D)

    BE = 3200
    wijm = pl.pallas_call(
        _filter_body,
        grid=(E // BE,),
        in_specs=[pl.BlockSpec((BE, 16), lambda i: (i, 0)),
                  pl.BlockSpec((BE, 1), lambda i: (i, 0)),
                  pl.BlockSpec((16, D), lambda i: (0, 0)),
                  pl.BlockSpec((1, D), lambda i: (0, 0)),
                  pl.BlockSpec((D, D), lambda i: (0, 0)),
                  pl.BlockSpec((1, D), lambda i: (0, 0))],
        out_specs=pl.BlockSpec((BE, D), lambda i: (i, 0)),
        out_shape=jax.ShapeDtypeStruct((E, D), jnp.float32),
    )(f_ij, rcut_ij.reshape(E, 1), fw1, fb1.reshape(1, D),
      fw2, fb2.reshape(1, D))

    gidx = jnp.stack([jj, ii + N]).reshape(2, NTILES, NCH, K)
    scat = jnp.stack([ii, jj]).reshape(2, NTILES, NCH, K)
    cidx = jnp.stack([gidx, scat], axis=3)
    zeros = jnp.zeros((ZPT, D), jnp.float32)

    conv3 = _sc_conv(table, cidx, wijm, zeros)[:, :N, :]
    w1s = jnp.stack([ow1, oyw1])
    b1s = jnp.stack([ob1, oyb1]).reshape(2, 1, D)
    w2s = jnp.stack([ow2, oyw2])
    b2s = jnp.stack([ob2, oyb2]).reshape(2, 1, D)
    outs = pl.pallas_call(
        _out_body,
        grid=(2, N // BN),
        in_specs=[pl.BlockSpec((1, BN, D), lambda g, i: (g, i, 0)),
                  pl.BlockSpec((1, D, D), lambda g, i: (g, 0, 0)),
                  pl.BlockSpec((1, 1, D), lambda g, i: (g, 0, 0)),
                  pl.BlockSpec((1, D, D), lambda g, i: (g, 0, 0)),
                  pl.BlockSpec((1, 1, D), lambda g, i: (g, 0, 0))],
        out_specs=pl.BlockSpec((1, BN, D), lambda g, i: (g, i, 0)),
        out_shape=jax.ShapeDtypeStruct((2, N, D), jnp.float32),
    )(conv3, w1s, b1s, w2s, b2s)
    return outs[0], outs[1]


# speedup vs baseline: 1.6471x; 1.6471x over previous
import functools

import jax
import jax.numpy as jnp
from jax import lax
from jax.experimental import pallas as pl
from jax.experimental.pallas import tpu as pltpu
from jax.experimental.pallas import tpu_sc as plsc

N = 10000
E = 320000
D = 128
NTILES = 16
EPT = E // NTILES
K = 80
NCH = EPT // K
HK = 40
H = EPT // HK
ACCR = 10240
ZPT = ACCR // NTILES
CPT = N // NTILES
_LN2 = 0.6931471805599453


def _ssp(t):
    return jnp.maximum(t, 0.0) + jnp.log1p(jnp.exp(-jnp.abs(t))) - _LN2


def _proj_body(a_ref, w_ref, o_ref):
    o_ref[...] = jnp.dot(a_ref[0], w_ref[0],
                         preferred_element_type=jnp.float32)[None]


def _filter_body(f_ref, rc_ref, w1_ref, b1_ref, w2_ref, b2_ref, o_ref):
    h = _ssp(jnp.dot(f_ref[...], w1_ref[...],
                     preferred_element_type=jnp.float32) + b1_ref[...])
    w = jnp.dot(h, w2_ref[...], preferred_element_type=jnp.float32) + b2_ref[...]
    o_ref[...] = w * rc_ref[...]


def _out_body(c_ref, w1_ref, b1_ref, w2_ref, b2_ref, o_ref):
    h = _ssp(jnp.dot(c_ref[0], w1_ref[0],
                     preferred_element_type=jnp.float32) + b1_ref[0])
    o_ref[...] = (jnp.dot(h, w2_ref[0],
                          preferred_element_type=jnp.float32) + b2_ref[0])[None]


_sc_mesh = plsc.VectorSubcoreMesh(core_axis_name="c", subcore_axis_name="s")


@functools.partial(
    pl.kernel,
    mesh=_sc_mesh,
    out_type=jax.ShapeDtypeStruct((2, ACCR, D), jnp.float32),
    scratch_types=[
        pltpu.VMEM((4, 2, HK), jnp.int32),
        pltpu.VMEM((2, HK, D), jnp.float32),
        pltpu.VMEM((2, HK, D), jnp.float32),
        pltpu.VMEM_SHARED((ACCR, D), jnp.float32),
        pltpu.SemaphoreType.DMA,
        pltpu.SemaphoreType.DMA,
        pltpu.SemaphoreType.DMA,
        pltpu.SemaphoreType.DMA,
        pltpu.SemaphoreType.DMA,
    ],
)
def _sc_conv(table, cidx, wij, zeros, out,
             cidx_v, wij_v, rows_v, acc,
             semg0, semg1, semw0, semw1, semc):
    c = lax.axis_index("c")
    s = lax.axis_index("s")
    semg = (semg0, semg1)
    semw = (semw0, semw1)
    pltpu.sync_copy(zeros, acc.at[pl.ds(s * ZPT, ZPT)])
    pltpu.sync_copy(cidx.at[c, s, 0], cidx_v.at[0])
    plsc.subcore_barrier()

    ebase = s * EPT

    pltpu.async_copy(cidx.at[c, s, 1], cidx_v.at[1], semc)
    pltpu.async_copy(wij.at[pl.ds(ebase, HK)], wij_v.at[0], semw0)
    pltpu.async_copy(table.at[cidx_v.at[0, 0]], rows_v.at[0], semg0)

    def pair(g2, carry):
        for b in (0, 1):
            nb = 1 - b
            g = g2 * 2 + b
            gn = jnp.minimum(g + 1, H - 1)
            gp = jnp.minimum(g + 2, H - 1)
            pltpu.make_async_copy(cidx.at[c, s, gn],
                                  cidx_v.at[lax.rem(g + 1, 4)], semc).wait()
            pltpu.async_copy(cidx.at[c, s, gp],
                             cidx_v.at[lax.rem(g + 2, 4)], semc)
            pltpu.async_copy(wij.at[pl.ds(ebase + gn * HK, HK)],
                             wij_v.at[nb], semw[nb])
            pltpu.async_copy(table.at[cidx_v.at[lax.rem(gn, 4), 0]],
                             rows_v.at[nb], semg[nb])
            pltpu.make_async_copy(wij.at[pl.ds(ebase, HK)],
                                  wij_v.at[b], semw[b]).wait()
            pltpu.make_async_copy(table.at[cidx_v.at[0, 0]],
                                  rows_v.at[b], semg[b]).wait()

            def emul(e4, cc):
                for ee in range(4):
                    for dd in range(D // 16):
                        sl = pl.ds(dd * 16, 16)
                        e = e4 * 4 + ee
                        rows_v[b, e, sl] = rows_v[b, e, sl] * wij_v[b, e, sl]
                return cc
            lax.fori_loop(0, HK // 4, emul, 0)
            pltpu.sync_copy(rows_v.at[b],
                            acc.at[cidx_v.at[lax.rem(g, 4), 1]], add=True)
        return carry

    lax.fori_loop(0, H // 2, pair, 0)
    pltpu.make_async_copy(cidx.at[c, s, 0], cidx_v.at[lax.rem(H + 1, 4)],
                          semc).wait()
    pltpu.make_async_copy(wij.at[pl.ds(ebase, HK)], wij_v.at[0], semw0).wait()
    pltpu.make_async_copy(table.at[cidx_v.at[0, 0]], rows_v.at[0],
                          semg0).wait()
    plsc.subcore_barrier()
    pltpu.sync_copy(acc.at[pl.ds(s * ZPT, ZPT)],
                    out.at[c, pl.ds(s * ZPT, ZPT)])


def kernel(x, y, f_ij, idx_i, idx_j, rcut_ij,
           W_in2f, W_in2f_y, fw1, fb1, fw2, fb2,
           ow1, ob1, ow2, ob2, oyw1, oyb1, oyw2, oyb2):
    perm = (jnp.arange(NTILES)[:, None, None] * EPT
            + jnp.arange(NCH)[None, :, None]
            + jnp.arange(K)[None, None, :] * NCH).reshape(E)
    f_ij = f_ij[perm]
    rcut_ij = rcut_ij[perm]
    ii = idx_i[perm].astype(jnp.int32)
    jj = idx_j[perm].astype(jnp.int32)

    BN = 2000
    ab = jnp.stack([y, x])
    Wb = jnp.stack([W_in2f_y, W_in2f])
    table = pl.pallas_call(
        _proj_body,
        grid=(2, N // BN),
        in_specs=[pl.BlockSpec((1, BN, D), lambda g, i: (g, i, 0)),
                  pl.BlockSpec((1, D, D), lambda g, i: (g, 0, 0))],
        out_specs=pl.BlockSpec((1, BN, D), lambda g, i: (g, i, 0)),
        out_shape=jax.ShapeDtypeStruct((2, N, D), jnp.float32),
    )(ab, Wb).reshape(2 * N, D)

    BE = 3200
    wijm = pl.pallas_call(
        _filter_body,
        grid=(E // BE,),
        in_specs=[pl.BlockSpec((BE, 16), lambda i: (i, 0)),
                  pl.BlockSpec((BE, 1), lambda i: (i, 0)),
                  pl.BlockSpec((16, D), lambda i: (0, 0)),
                  pl.BlockSpec((1, D), lambda i: (0, 0)),
                  pl.BlockSpec((D, D), lambda i: (0, 0)),
                  pl.BlockSpec((1, D), lambda i: (0, 0))],
        out_specs=pl.BlockSpec((BE, D), lambda i: (i, 0)),
        out_shape=jax.ShapeDtypeStruct((E, D), jnp.float32),
    )(f_ij, rcut_ij.reshape(E, 1), fw1, fb1.reshape(1, D),
      fw2, fb2.reshape(1, D))

    gidx = jnp.stack([jj, ii + N]).reshape(2, NTILES, H, HK)
    scat = jnp.stack([ii, jj]).reshape(2, NTILES, H, HK)
    cidx = jnp.stack([gidx, scat], axis=3)
    zeros = jnp.zeros((ZPT, D), jnp.float32)

    conv3 = _sc_conv(table, cidx, wijm, zeros)[:, :N, :]
    w1s = jnp.stack([ow1, oyw1])
    b1s = jnp.stack([ob1, oyb1]).reshape(2, 1, D)
    w2s = jnp.stack([ow2, oyw2])
    b2s = jnp.stack([ob2, oyb2]).reshape(2, 1, D)
    outs = pl.pallas_call(
        _out_body,
        grid=(2, N // BN),
        in_specs=[pl.BlockSpec((1, BN, D), lambda g, i: (g, i, 0)),
                  pl.BlockSpec((1, D, D), lambda g, i: (g, 0, 0)),
                  pl.BlockSpec((1, 1, D), lambda g, i: (g, 0, 0)),
                  pl.BlockSpec((1, D, D), lambda g, i: (g, 0, 0)),
                  pl.BlockSpec((1, 1, D), lambda g, i: (g, 0, 0))],
        out_specs=pl.BlockSpec((1, BN, D), lambda g, i: (g, i, 0)),
        out_shape=jax.ShapeDtypeStruct((2, N, D), jnp.float32),
    )(conv3, w1s, b1s, w2s, b2s)
    return outs[0], outs[1]
